# baseline (speedup 1.0000x reference)
import jax
import jax.numpy as jnp
from jax.experimental import pallas as pl
from jax.experimental.pallas import tpu as pltpu

NUM_Q_HEADS = 32
NUM_KV_HEADS = 8
HEAD_DIM = 128
GQA = NUM_Q_HEADS // NUM_KV_HEADS
SCALE = HEAD_DIM ** -0.5
NUM_BLOCKS = 2048
BLOCK_SIZE = 16
B = 16
MAX_BLOCKS_PER_SEQ = 128
FUSED = NUM_KV_HEADS * HEAD_DIM
NEG_INF = float("-inf")


def _attn_body(nb_ref, bt_ref, sl_ref, sm_ref,
               qbd_ref, k_new_ref, v_new_ref, mask_ref, kc_ref, vc_ref,
               out_ref,
               m_ref, l_ref, acc_ref):
    b = pl.program_id(0)
    i = pl.program_id(1)
    nb = nb_ref[b]

    @pl.when(i == 0)
    def _init():
        m_ref[...] = jnp.full_like(m_ref, NEG_INF)
        l_ref[...] = jnp.zeros_like(l_ref)
        acc_ref[...] = jnp.zeros_like(acc_ref)

    @pl.when(i < nb)
    def _compute():
        pb = bt_ref[b, i]
        for w in range(B):
            slot = sm_ref[w]
            off = jax.lax.rem(slot, BLOCK_SIZE)

            @pl.when(jax.lax.div(slot, BLOCK_SIZE) == pb)
            def _patch():
                kc_ref[0, pl.ds(off, 1)] = k_new_ref[w : w + 1]
                vc_ref[0, pl.ds(off, 1)] = v_new_ref[w : w + 1]

        kb = kc_ref[0]
        vb = vc_ref[0]
        s = jax.lax.dot_general(
            qbd_ref[0], kb, (((1,), (1,)), ((), ())),
            preferred_element_type=jnp.float32) * SCALE
        rem = sl_ref[b] - i * BLOCK_SIZE
        lane = jax.lax.broadcasted_iota(jnp.int32, (NUM_Q_HEADS, BLOCK_SIZE), 1)
        s = jnp.where(lane < rem, s, NEG_INF)
        m_old = m_ref[...]
        m_new = jnp.maximum(m_old, jnp.max(s, axis=1, keepdims=True))
        alpha = jnp.exp(m_old - m_new)
        p = jnp.exp(s - m_new)
        l_ref[...] = alpha * l_ref[...] + jnp.sum(p, axis=1, keepdims=True)
        pv = jax.lax.dot_general(
            p, vb, (((1,), (0,)), ((), ())),
            preferred_element_type=jnp.float32)
        acc_ref[...] = alpha * acc_ref[...] + pv
        m_ref[...] = m_new

    @pl.when(i == nb - 1)
    def _finalize():
        a = acc_ref[...] * mask_ref[...]
        o = a[:, 0:HEAD_DIM]
        for j in range(1, NUM_KV_HEADS):
            o = o + a[:, j * HEAD_DIM : (j + 1) * HEAD_DIM]
        out_ref[0] = o / l_ref[...]


def _kv_index_map(b, i, nb_ref, bt_ref, sl_ref, sm_ref):
    return (bt_ref[b, jnp.minimum(i, nb_ref[b] - 1)], 0, 0)


@jax.jit
def _paged_attn(q, k, v, k_cache, v_cache, slot_mapping, block_tables,
                seq_lens):
    nb = (seq_lens + BLOCK_SIZE - 1) // BLOCK_SIZE
    kc3 = k_cache.reshape(NUM_BLOCKS, BLOCK_SIZE, FUSED)
    vc3 = v_cache.reshape(NUM_BLOCKS, BLOCK_SIZE, FUSED)
    k2 = k.reshape(B, FUSED)
    v2 = v.reshape(B, FUSED)
    bd = (jnp.arange(FUSED)[None, :] // HEAD_DIM
          == jnp.arange(NUM_Q_HEADS)[:, None] // GQA)
    bd = bd.astype(jnp.float32)
    q_bd = jnp.tile(q, (1, 1, NUM_KV_HEADS)) * bd[None]

    grid_spec = pltpu.PrefetchScalarGridSpec(
        num_scalar_prefetch=4,
        grid=(B, MAX_BLOCKS_PER_SEQ),
        in_specs=[
            pl.BlockSpec((1, NUM_Q_HEADS, FUSED), lambda b, i, *_: (b, 0, 0)),
            pl.BlockSpec((B, FUSED), lambda b, i, *_: (0, 0)),
            pl.BlockSpec((B, FUSED), lambda b, i, *_: (0, 0)),
            pl.BlockSpec((NUM_Q_HEADS, FUSED), lambda b, i, *_: (0, 0)),
            pl.BlockSpec((1, BLOCK_SIZE, FUSED), _kv_index_map),
            pl.BlockSpec((1, BLOCK_SIZE, FUSED), _kv_index_map),
        ],
        out_specs=pl.BlockSpec((1, NUM_Q_HEADS, HEAD_DIM),
                               lambda b, i, *_: (b, 0, 0)),
        scratch_shapes=[
            pltpu.VMEM((NUM_Q_HEADS, 1), jnp.float32),
            pltpu.VMEM((NUM_Q_HEADS, 1), jnp.float32),
            pltpu.VMEM((NUM_Q_HEADS, FUSED), jnp.float32),
        ],
    )
    return pl.pallas_call(
        _attn_body,
        grid_spec=grid_spec,
        out_shape=jax.ShapeDtypeStruct((B, NUM_Q_HEADS, HEAD_DIM),
                                       jnp.float32),
        compiler_params=pltpu.CompilerParams(
            dimension_semantics=("arbitrary", "arbitrary")),
    )(nb, block_tables, seq_lens, slot_mapping, q_bd, k2, v2, bd, kc3, vc3)


def kernel(q, k, v, k_cache, v_cache, slot_mapping, block_tables, seq_lens,
           query_lens, is_prefill):
    del query_lens, is_prefill
    return _paged_attn(q, k, v, k_cache, v_cache, slot_mapping, block_tables,
                       seq_lens)


# speedup vs baseline: 1.8863x; 1.8863x over previous
import jax
import jax.numpy as jnp
from jax.experimental import pallas as pl
from jax.experimental.pallas import tpu as pltpu

NUM_Q_HEADS = 32
NUM_KV_HEADS = 8
HEAD_DIM = 128
GQA = NUM_Q_HEADS // NUM_KV_HEADS
SCALE = HEAD_DIM ** -0.5
NUM_BLOCKS = 2048
BLOCK_SIZE = 16
B = 16
MAX_BLOCKS_PER_SEQ = 128
FUSED = NUM_KV_HEADS * HEAD_DIM
CHUNK = 8
CHUNK_TOK = CHUNK * BLOCK_SIZE
NUM_CHUNKS = MAX_BLOCKS_PER_SEQ // CHUNK
NEG_INF = float("-inf")


def _attn_body(nb_ref, bt_ref, sl_ref, sm_ref, pw_ref,
               qbd_ref, k_new_ref, v_new_ref, mask_ref,
               *rest):
    kc_refs = rest[0:CHUNK]
    vc_refs = rest[CHUNK:2 * CHUNK]
    out_ref = rest[2 * CHUNK]
    m_ref, l_ref, acc_ref, kcat_ref, vcat_ref = rest[2 * CHUNK + 1:]

    b = pl.program_id(0)
    i = pl.program_id(1)
    nb = nb_ref[b]

    @pl.when(jnp.logical_and(b == 0, i == 0))
    def _init_vcat():
        vcat_ref[...] = jnp.zeros_like(vcat_ref)

    @pl.when(i == 0)
    def _init():
        m_ref[...] = jnp.full_like(m_ref, NEG_INF)
        l_ref[...] = jnp.zeros_like(l_ref)
        acc_ref[...] = jnp.zeros_like(acc_ref)

    @pl.when(i * CHUNK < nb)
    def _compute():
        for j in range(CHUNK):
            @pl.when(i * CHUNK + j < nb)
            def _copy():
                kcat_ref[pl.ds(j * BLOCK_SIZE, BLOCK_SIZE)] = kc_refs[j][0]
                vcat_ref[pl.ds(j * BLOCK_SIZE, BLOCK_SIZE)] = vc_refs[j][0]
                bits = pw_ref[b, i * CHUNK + j]

                @pl.when(bits != 0)
                def _patch_any():
                    for w in range(B):
                        @pl.when((jax.lax.shift_right_logical(bits, w) & 1)
                                 == 1)
                        def _patch():
                            off = (j * BLOCK_SIZE
                                   + jax.lax.rem(sm_ref[w], BLOCK_SIZE))
                            kcat_ref[pl.ds(off, 1)] = k_new_ref[w : w + 1]
                            vcat_ref[pl.ds(off, 1)] = v_new_ref[w : w + 1]

        s = jax.lax.dot_general(
            qbd_ref[0], kcat_ref[...], (((1,), (1,)), ((), ())),
            preferred_element_type=jnp.float32) * SCALE
        rem = sl_ref[b] - i * CHUNK_TOK
        lane = jax.lax.broadcasted_iota(jnp.int32, (NUM_Q_HEADS, CHUNK_TOK), 1)
        s = jnp.where(lane < rem, s, NEG_INF)
        m_old = m_ref[...]
        m_new = jnp.maximum(m_old, jnp.max(s, axis=1, keepdims=True))
        alpha = jnp.exp(m_old - m_new)
        p = jnp.exp(s - m_new)
        l_ref[...] = alpha * l_ref[...] + jnp.sum(p, axis=1, keepdims=True)
        pv = jax.lax.dot_general(
            p, vcat_ref[...], (((1,), (0,)), ((), ())),
            preferred_element_type=jnp.float32)
        acc_ref[...] = alpha * acc_ref[...] + pv
        m_ref[...] = m_new

    @pl.when(i == (nb - 1) // CHUNK)
    def _finalize():
        a = acc_ref[...] * mask_ref[...]
        o = a[:, 0:HEAD_DIM]
        for j in range(1, NUM_KV_HEADS):
            o = o + a[:, j * HEAD_DIM : (j + 1) * HEAD_DIM]
        out_ref[0] = o / l_ref[...]


def _make_kv_index_map(j):
    def _map(b, i, nb_ref, bt_ref, sl_ref, sm_ref, pw_ref):
        nb = nb_ref[b]
        pj = j + CHUNK * jnp.maximum(0, (nb - 1 - j) // CHUNK)
        pos = jnp.minimum(i * CHUNK + j, pj)
        return (bt_ref[b, pos], 0, 0)
    return _map


@jax.jit
def _paged_attn(q, k, v, k_cache, v_cache, slot_mapping, block_tables,
                seq_lens):
    nb = (seq_lens + BLOCK_SIZE - 1) // BLOCK_SIZE
    kc3 = k_cache.reshape(NUM_BLOCKS, BLOCK_SIZE, FUSED)
    vc3 = v_cache.reshape(NUM_BLOCKS, BLOCK_SIZE, FUSED)
    k2 = k.reshape(B, FUSED)
    v2 = v.reshape(B, FUSED)
    bd = (jnp.arange(FUSED)[None, :] // HEAD_DIM
          == jnp.arange(NUM_Q_HEADS)[:, None] // GQA)
    bd = bd.astype(jnp.float32)
    q_bd = jnp.tile(q, (1, 1, NUM_KV_HEADS)) * bd[None]
    slot_blk = slot_mapping.astype(jnp.int32) // BLOCK_SIZE
    hits = block_tables[:, :, None] == slot_blk[None, None, :]
    pw = jnp.sum(jnp.where(hits, jnp.int32(1) << jnp.arange(B, dtype=jnp.int32)[None, None, :], 0),
                 axis=-1).astype(jnp.int32)

    kv_specs = [pl.BlockSpec((1, BLOCK_SIZE, FUSED), _make_kv_index_map(j))
                for j in range(CHUNK)]
    grid_spec = pltpu.PrefetchScalarGridSpec(
        num_scalar_prefetch=5,
        grid=(B, NUM_CHUNKS),
        in_specs=[
            pl.BlockSpec((1, NUM_Q_HEADS, FUSED), lambda b, i, *_: (b, 0, 0)),
            pl.BlockSpec((B, FUSED), lambda b, i, *_: (0, 0)),
            pl.BlockSpec((B, FUSED), lambda b, i, *_: (0, 0)),
            pl.BlockSpec((NUM_Q_HEADS, FUSED), lambda b, i, *_: (0, 0)),
        ] + kv_specs + kv_specs,
        out_specs=pl.BlockSpec((1, NUM_Q_HEADS, HEAD_DIM),
                               lambda b, i, *_: (b, 0, 0)),
        scratch_shapes=[
            pltpu.VMEM((NUM_Q_HEADS, 1), jnp.float32),
            pltpu.VMEM((NUM_Q_HEADS, 1), jnp.float32),
            pltpu.VMEM((NUM_Q_HEADS, FUSED), jnp.float32),
            pltpu.VMEM((CHUNK_TOK, FUSED), jnp.float32),
            pltpu.VMEM((CHUNK_TOK, FUSED), jnp.float32),
        ],
    )
    return pl.pallas_call(
        _attn_body,
        grid_spec=grid_spec,
        out_shape=jax.ShapeDtypeStruct((B, NUM_Q_HEADS, HEAD_DIM),
                                       jnp.float32),
        compiler_params=pltpu.CompilerParams(
            dimension_semantics=("arbitrary", "arbitrary")),
    )(nb, block_tables, seq_lens, slot_mapping, pw,
      q_bd, k2, v2, bd, *([kc3] * CHUNK), *([vc3] * CHUNK))


def kernel(q, k, v, k_cache, v_cache, slot_mapping, block_tables, seq_lens,
           query_lens, is_prefill):
    del query_lens, is_prefill
    return _paged_attn(q, k, v, k_cache, v_cache, slot_mapping, block_tables,
                       seq_lens)


# speedup vs baseline: 2.8871x; 1.5306x over previous
import jax
import jax.numpy as jnp
from jax.experimental import pallas as pl
from jax.experimental.pallas import tpu as pltpu

NUM_Q_HEADS = 32
NUM_KV_HEADS = 8
HEAD_DIM = 128
GQA = NUM_Q_HEADS // NUM_KV_HEADS
SCALE = HEAD_DIM ** -0.5
NUM_BLOCKS = 2048
BLOCK_SIZE = 16
B = 16
MAX_BLOCKS_PER_SEQ = 128
FUSED = NUM_KV_HEADS * HEAD_DIM
CHUNK = 16
CHUNK_TOK = CHUNK * BLOCK_SIZE
MAX_CHUNKS = MAX_BLOCKS_PER_SEQ // CHUNK
NBUF = 3
AHEAD = NBUF - 1
NEG_INF = float("-inf")


def _attn_body(nb_ref, bt_ref, sl_ref, sm_ref, pw_ref, ca_ref,
               qbd_ref, k_new_ref, v_new_ref, mask_ref, kc_hbm, vc_hbm,
               out_ref,
               m_ref, l_ref, acc_ref, kcat_ref, vcat_ref, sem_ref):
    b = pl.program_id(0)
    nb = nb_ref[b]
    nchunks = (nb + CHUNK - 1) // CHUNK

    def _issue(c):
        slot = jax.lax.rem(c, NBUF)
        for j in range(CHUNK):
            pos = jnp.minimum(c * CHUNK + j, nb - 1)
            pb = bt_ref[b, pos]
            pltpu.make_async_copy(
                kc_hbm.at[pb],
                kcat_ref.at[slot, pl.ds(j * BLOCK_SIZE, BLOCK_SIZE)],
                sem_ref.at[slot]).start()
            pltpu.make_async_copy(
                vc_hbm.at[pb],
                vcat_ref.at[slot, pl.ds(j * BLOCK_SIZE, BLOCK_SIZE)],
                sem_ref.at[slot]).start()

    @pl.when(b == 0)
    def _init_vcat():
        vcat_ref[...] = jnp.zeros_like(vcat_ref)

    m_ref[...] = jnp.full_like(m_ref, NEG_INF)
    l_ref[...] = jnp.zeros_like(l_ref)
    acc_ref[...] = jnp.zeros_like(acc_ref)

    jax.lax.fori_loop(0, jnp.minimum(AHEAD, nchunks),
                      lambda c, _: (_issue(c), 0)[1], 0)

    def _chunk_body(c, _):
        @pl.when(c + AHEAD < nchunks)
        def _issue_ahead():
            _issue(c + AHEAD)

        slot = jax.lax.rem(c, NBUF)
        for j in range(CHUNK):
            pltpu.make_async_copy(
                kc_hbm.at[bt_ref[b, 0]],
                kcat_ref.at[slot, pl.ds(j * BLOCK_SIZE, BLOCK_SIZE)],
                sem_ref.at[slot]).wait()
            pltpu.make_async_copy(
                vc_hbm.at[bt_ref[b, 0]],
                vcat_ref.at[slot, pl.ds(j * BLOCK_SIZE, BLOCK_SIZE)],
                sem_ref.at[slot]).wait()

        @pl.when(ca_ref[b, c] != 0)
        def _patch_chunk():
            for j in range(CHUNK):
                bits = pw_ref[b, c * CHUNK + j]

                @pl.when(bits != 0)
                def _patch_block():
                    for w in range(B):
                        @pl.when((jax.lax.shift_right_logical(bits, w) & 1)
                                 == 1)
                        def _patch():
                            off = (j * BLOCK_SIZE
                                   + jax.lax.rem(sm_ref[w], BLOCK_SIZE))
                            kcat_ref[slot, pl.ds(off, 1)] = \
                                k_new_ref[w : w + 1]
                            vcat_ref[slot, pl.ds(off, 1)] = \
                                v_new_ref[w : w + 1]

        kc = kcat_ref[slot]
        vc = vcat_ref[slot]
        s = jax.lax.dot_general(
            qbd_ref[0], kc, (((1,), (1,)), ((), ())),
            preferred_element_type=jnp.float32) * SCALE
        rem = sl_ref[b] - c * CHUNK_TOK
        lane = jax.lax.broadcasted_iota(jnp.int32, (NUM_Q_HEADS, CHUNK_TOK), 1)
        s = jnp.where(lane < rem, s, NEG_INF)
        m_old = m_ref[...]
        m_new = jnp.maximum(m_old, jnp.max(s, axis=1, keepdims=True))
        alpha = jnp.exp(m_old - m_new)
        p = jnp.exp(s - m_new)
        l_ref[...] = alpha * l_ref[...] + jnp.sum(p, axis=1, keepdims=True)
        pv = jax.lax.dot_general(
            p, vc, (((1,), (0,)), ((), ())),
            preferred_element_type=jnp.float32)
        acc_ref[...] = alpha * acc_ref[...] + pv
        m_ref[...] = m_new
        return 0

    jax.lax.fori_loop(0, nchunks, _chunk_body, 0)

    a = acc_ref[...] * mask_ref[...]
    o = a[:, 0:HEAD_DIM]
    for j in range(1, NUM_KV_HEADS):
        o = o + a[:, j * HEAD_DIM : (j + 1) * HEAD_DIM]
    out_ref[0] = o / l_ref[...]


@jax.jit
def _paged_attn(q, k, v, k_cache, v_cache, slot_mapping, block_tables,
                seq_lens):
    nb = (seq_lens + BLOCK_SIZE - 1) // BLOCK_SIZE
    kc3 = k_cache.reshape(NUM_BLOCKS, BLOCK_SIZE, FUSED)
    vc3 = v_cache.reshape(NUM_BLOCKS, BLOCK_SIZE, FUSED)
    k2 = k.reshape(B, FUSED)
    v2 = v.reshape(B, FUSED)
    bd = (jnp.arange(FUSED)[None, :] // HEAD_DIM
          == jnp.arange(NUM_Q_HEADS)[:, None] // GQA)
    bd = bd.astype(jnp.float32)
    q_bd = jnp.tile(q, (1, 1, NUM_KV_HEADS)) * bd[None]
    slot_blk = slot_mapping.astype(jnp.int32) // BLOCK_SIZE
    hits = block_tables[:, :, None] == slot_blk[None, None, :]
    pw = jnp.sum(jnp.where(hits,
                           jnp.int32(1) << jnp.arange(B, dtype=jnp.int32)[None, None, :],
                           0), axis=-1).astype(jnp.int32)
    ca = jnp.sum(pw.reshape(B, MAX_CHUNKS, CHUNK), axis=-1)

    grid_spec = pltpu.PrefetchScalarGridSpec(
        num_scalar_prefetch=6,
        grid=(B,),
        in_specs=[
            pl.BlockSpec((1, NUM_Q_HEADS, FUSED), lambda b, *_: (b, 0, 0)),
            pl.BlockSpec((B, FUSED), lambda b, *_: (0, 0)),
            pl.BlockSpec((B, FUSED), lambda b, *_: (0, 0)),
            pl.BlockSpec((NUM_Q_HEADS, FUSED), lambda b, *_: (0, 0)),
            pl.BlockSpec(memory_space=pltpu.MemorySpace.HBM),
            pl.BlockSpec(memory_space=pltpu.MemorySpace.HBM),
        ],
        out_specs=pl.BlockSpec((1, NUM_Q_HEADS, HEAD_DIM),
                               lambda b, *_: (b, 0, 0)),
        scratch_shapes=[
            pltpu.VMEM((NUM_Q_HEADS, 1), jnp.float32),
            pltpu.VMEM((NUM_Q_HEADS, 1), jnp.float32),
            pltpu.VMEM((NUM_Q_HEADS, FUSED), jnp.float32),
            pltpu.VMEM((NBUF, CHUNK_TOK, FUSED), jnp.float32),
            pltpu.VMEM((NBUF, CHUNK_TOK, FUSED), jnp.float32),
            pltpu.SemaphoreType.DMA((NBUF,)),
        ],
    )
    return pl.pallas_call(
        _attn_body,
        grid_spec=grid_spec,
        out_shape=jax.ShapeDtypeStruct((B, NUM_Q_HEADS, HEAD_DIM),
                                       jnp.float32),
        compiler_params=pltpu.CompilerParams(
            dimension_semantics=("arbitrary",)),
    )(nb, block_tables, seq_lens, slot_mapping, pw, ca,
      q_bd, k2, v2, bd, kc3, vc3)


def kernel(q, k, v, k_cache, v_cache, slot_mapping, block_tables, seq_lens,
           query_lens, is_prefill):
    del query_lens, is_prefill
    return _paged_attn(q, k, v, k_cache, v_cache, slot_mapping, block_tables,
                       seq_lens)


# speedup vs baseline: 2.9339x; 1.0162x over previous
import jax
import jax.numpy as jnp
from jax.experimental import pallas as pl
from jax.experimental.pallas import tpu as pltpu

NUM_Q_HEADS = 32
NUM_KV_HEADS = 8
HEAD_DIM = 128
GQA = NUM_Q_HEADS // NUM_KV_HEADS
SCALE = HEAD_DIM ** -0.5
NUM_BLOCKS = 2048
BLOCK_SIZE = 16
B = 16
MAX_BLOCKS_PER_SEQ = 128
FUSED = NUM_KV_HEADS * HEAD_DIM
CHUNK = 16
CHUNK_TOK = CHUNK * BLOCK_SIZE
MAX_CHUNKS = MAX_BLOCKS_PER_SEQ // CHUNK
NBUF = 4
AHEAD = NBUF - 1
NEG_INF = float("-inf")


def _attn_body(nb_ref, bt_ref, sl_ref, sm_ref, pw_ref, ca_ref,
               qbd_ref, k_new_ref, v_new_ref, mask_ref, kc_hbm, vc_hbm,
               out_ref,
               m_ref, l_ref, acc_ref, kcat_ref, vcat_ref, sem_ref):
    b = pl.program_id(0)
    nb = nb_ref[b]
    nchunks = (nb + CHUNK - 1) // CHUNK

    def _issue(c):
        slot = jax.lax.rem(c, NBUF)
        for j in range(CHUNK):
            pos = jnp.minimum(c * CHUNK + j, nb - 1)
            pb = bt_ref[b, pos]
            pltpu.make_async_copy(
                kc_hbm.at[pb],
                kcat_ref.at[slot, pl.ds(j * BLOCK_SIZE, BLOCK_SIZE)],
                sem_ref.at[slot]).start()
            pltpu.make_async_copy(
                vc_hbm.at[pb],
                vcat_ref.at[slot, pl.ds(j * BLOCK_SIZE, BLOCK_SIZE)],
                sem_ref.at[slot]).start()

    m_ref[...] = jnp.full_like(m_ref, NEG_INF)
    l_ref[...] = jnp.zeros_like(l_ref)
    acc_ref[...] = jnp.zeros_like(acc_ref)

    jax.lax.fori_loop(0, jnp.minimum(AHEAD, nchunks),
                      lambda c, _: (_issue(c), 0)[1], 0)

    def _chunk_body(c, _):
        @pl.when(c + AHEAD < nchunks)
        def _issue_ahead():
            _issue(c + AHEAD)

        slot = jax.lax.rem(c, NBUF)
        for j in range(CHUNK):
            pltpu.make_async_copy(
                kc_hbm.at[bt_ref[b, 0]],
                kcat_ref.at[slot, pl.ds(j * BLOCK_SIZE, BLOCK_SIZE)],
                sem_ref.at[slot]).wait()
            pltpu.make_async_copy(
                vc_hbm.at[bt_ref[b, 0]],
                vcat_ref.at[slot, pl.ds(j * BLOCK_SIZE, BLOCK_SIZE)],
                sem_ref.at[slot]).wait()

        @pl.when(ca_ref[b, c] != 0)
        def _patch_chunk():
            for j in range(CHUNK):
                bits = pw_ref[b, c * CHUNK + j]

                @pl.when(bits != 0)
                def _patch_block():
                    for w in range(B):
                        @pl.when((jax.lax.shift_right_logical(bits, w) & 1)
                                 == 1)
                        def _patch():
                            off = (j * BLOCK_SIZE
                                   + jax.lax.rem(sm_ref[w], BLOCK_SIZE))
                            kcat_ref[slot, pl.ds(off, 1)] = \
                                k_new_ref[w : w + 1]
                            vcat_ref[slot, pl.ds(off, 1)] = \
                                v_new_ref[w : w + 1]

        kc = kcat_ref[slot]
        vc = vcat_ref[slot]
        s = jax.lax.dot_general(
            qbd_ref[0], kc, (((1,), (1,)), ((), ())),
            preferred_element_type=jnp.float32) * SCALE
        rem = sl_ref[b] - c * CHUNK_TOK
        lane = jax.lax.broadcasted_iota(jnp.int32, (NUM_Q_HEADS, CHUNK_TOK), 1)
        s = jnp.where(lane < rem, s, NEG_INF)
        m_old = m_ref[...]
        m_new = jnp.maximum(m_old, jnp.max(s, axis=1, keepdims=True))
        alpha = jnp.exp(m_old - m_new)
        p = jnp.exp(s - m_new)
        l_ref[...] = alpha * l_ref[...] + jnp.sum(p, axis=1, keepdims=True)
        pv = jax.lax.dot_general(
            p, vc, (((1,), (0,)), ((), ())),
            preferred_element_type=jnp.float32)
        acc_ref[...] = alpha * acc_ref[...] + pv
        m_ref[...] = m_new
        return 0

    jax.lax.fori_loop(0, nchunks, _chunk_body, 0)

    a = acc_ref[...] * mask_ref[...]
    o = a[:, 0:HEAD_DIM]
    for j in range(1, NUM_KV_HEADS):
        o = o + a[:, j * HEAD_DIM : (j + 1) * HEAD_DIM]
    out_ref[0] = o / l_ref[...]


@jax.jit
def _paged_attn(q, k, v, k_cache, v_cache, slot_mapping, block_tables,
                seq_lens):
    nb = (seq_lens + BLOCK_SIZE - 1) // BLOCK_SIZE
    kc3 = k_cache.reshape(NUM_BLOCKS, BLOCK_SIZE, FUSED)
    vc3 = v_cache.reshape(NUM_BLOCKS, BLOCK_SIZE, FUSED)
    k2 = k.reshape(B, FUSED)
    v2 = v.reshape(B, FUSED)
    bd = (jnp.arange(FUSED)[None, :] // HEAD_DIM
          == jnp.arange(NUM_Q_HEADS)[:, None] // GQA)
    bd = bd.astype(jnp.float32)
    q_bd = jnp.tile(q, (1, 1, NUM_KV_HEADS)) * bd[None]
    slot_blk = slot_mapping.astype(jnp.int32) // BLOCK_SIZE
    hits = block_tables[:, :, None] == slot_blk[None, None, :]
    pw = jnp.sum(jnp.where(hits,
                           jnp.int32(1) << jnp.arange(B, dtype=jnp.int32)[None, None, :],
                           0), axis=-1).astype(jnp.int32)
    ca = jnp.sum(pw.reshape(B, MAX_CHUNKS, CHUNK), axis=-1)

    grid_spec = pltpu.PrefetchScalarGridSpec(
        num_scalar_prefetch=6,
        grid=(B,),
        in_specs=[
            pl.BlockSpec((1, NUM_Q_HEADS, FUSED), lambda b, *_: (b, 0, 0)),
            pl.BlockSpec((B, FUSED), lambda b, *_: (0, 0)),
            pl.BlockSpec((B, FUSED), lambda b, *_: (0, 0)),
            pl.BlockSpec((NUM_Q_HEADS, FUSED), lambda b, *_: (0, 0)),
            pl.BlockSpec(memory_space=pltpu.MemorySpace.HBM),
            pl.BlockSpec(memory_space=pltpu.MemorySpace.HBM),
        ],
        out_specs=pl.BlockSpec((1, NUM_Q_HEADS, HEAD_DIM),
                               lambda b, *_: (b, 0, 0)),
        scratch_shapes=[
            pltpu.VMEM((NUM_Q_HEADS, 1), jnp.float32),
            pltpu.VMEM((NUM_Q_HEADS, 1), jnp.float32),
            pltpu.VMEM((NUM_Q_HEADS, FUSED), jnp.float32),
            pltpu.VMEM((NBUF, CHUNK_TOK, FUSED), jnp.float32),
            pltpu.VMEM((NBUF, CHUNK_TOK, FUSED), jnp.float32),
            pltpu.SemaphoreType.DMA((NBUF,)),
        ],
    )
    return pl.pallas_call(
        _attn_body,
        grid_spec=grid_spec,
        out_shape=jax.ShapeDtypeStruct((B, NUM_Q_HEADS, HEAD_DIM),
                                       jnp.float32),
        compiler_params=pltpu.CompilerParams(
            dimension_semantics=("arbitrary",)),
    )(nb, block_tables, seq_lens, slot_mapping, pw, ca,
      q_bd, k2, v2, bd, kc3, vc3)


def kernel(q, k, v, k_cache, v_cache, slot_mapping, block_tables, seq_lens,
           query_lens, is_prefill):
    del query_lens, is_prefill
    return _paged_attn(q, k, v, k_cache, v_cache, slot_mapping, block_tables,
                       seq_lens)
